# Initial kernel scaffold; baseline (speedup 1.0000x reference)
import jax
import jax.numpy as jnp
from jax.experimental import pallas as pl


def kernel(b):
    raise NotImplementedError("write your pallas kernel here")


# speedup vs baseline: 6.9077x; 6.9077x over previous
import jax
import jax.numpy as jnp
from jax.experimental import pallas as pl

_ROWS = 256 * 256
_Z = 256
_BLOCK_ROWS = 4096


def _copy_fix_body(x_ref, o_ref):
    x = x_ref[...]
    z = jax.lax.broadcasted_iota(jnp.int32, x.shape, 1)
    o_ref[...] = jnp.where(z == 0, x[:, 1:2], x)


def kernel(b):
    b2 = b.reshape(_ROWS, _Z)
    out = pl.pallas_call(
        _copy_fix_body,
        grid=(_ROWS // _BLOCK_ROWS,),
        in_specs=[pl.BlockSpec((_BLOCK_ROWS, _Z), lambda i: (i, 0))],
        out_specs=pl.BlockSpec((_BLOCK_ROWS, _Z), lambda i: (i, 0)),
        out_shape=jax.ShapeDtypeStruct((_ROWS, _Z), b.dtype),
    )(b2)
    return out.reshape(b.shape)


# speedup vs baseline: 6.9368x; 1.0042x over previous
import jax
import jax.numpy as jnp
from jax.experimental import pallas as pl

_ROWS = 256 * 256
_Z = 256
_BLOCK_ROWS = 8192


def _copy_fix_body(x_ref, o_ref):
    x = x_ref[...]
    z = jax.lax.broadcasted_iota(jnp.int32, x.shape, 1)
    o_ref[...] = jnp.where(z == 0, x[:, 1:2], x)


def kernel(b):
    b2 = b.reshape(_ROWS, _Z)
    out = pl.pallas_call(
        _copy_fix_body,
        grid=(_ROWS // _BLOCK_ROWS,),
        in_specs=[pl.BlockSpec((_BLOCK_ROWS, _Z), lambda i: (i, 0))],
        out_specs=pl.BlockSpec((_BLOCK_ROWS, _Z), lambda i: (i, 0)),
        out_shape=jax.ShapeDtypeStruct((_ROWS, _Z), b.dtype),
    )(b2)
    return out.reshape(b.shape)


# speedup vs baseline: 23.7683x; 3.4264x over previous
import jax
import jax.numpy as jnp
from jax.experimental import pallas as pl

_R = 131072
_C = 128
_BR = 8192


def _copy_fix_body(x_ref, o_ref):
    x = x_ref[...]
    row = jax.lax.broadcasted_iota(jnp.int32, x.shape, 0)
    col = jax.lax.broadcasted_iota(jnp.int32, x.shape, 1)
    fix = jnp.logical_and(col == 0, (row % 2) == 0)
    o_ref[...] = jnp.where(fix, x[:, 1:2], x)


def kernel(b):
    b2 = b.reshape(_R, _C)
    out = pl.pallas_call(
        _copy_fix_body,
        grid=(_R // _BR,),
        in_specs=[pl.BlockSpec((_BR, _C), lambda i: (i, 0))],
        out_specs=pl.BlockSpec((_BR, _C), lambda i: (i, 0)),
        out_shape=jax.ShapeDtypeStruct((_R, _C), b.dtype),
    )(b2)
    return out.reshape(b.shape)


# speedup vs baseline: 24.3907x; 1.0262x over previous
import jax
import jax.numpy as jnp
from jax.experimental import pallas as pl

_R = 131072
_C = 128
_BR = 16384


def _copy_fix_body(x_ref, o_ref):
    x = x_ref[...]
    row = jax.lax.broadcasted_iota(jnp.int32, x.shape, 0)
    col = jax.lax.broadcasted_iota(jnp.int32, x.shape, 1)
    fix = jnp.logical_and(col == 0, (row % 2) == 0)
    o_ref[...] = jnp.where(fix, x[:, 1:2], x)


def kernel(b):
    b2 = b.reshape(_R, _C)
    out = pl.pallas_call(
        _copy_fix_body,
        grid=(_R // _BR,),
        in_specs=[pl.BlockSpec((_BR, _C), lambda i: (i, 0))],
        out_specs=pl.BlockSpec((_BR, _C), lambda i: (i, 0)),
        out_shape=jax.ShapeDtypeStruct((_R, _C), b.dtype),
    )(b2)
    return out.reshape(b.shape)
